# baseline (speedup 1.0000x reference)
import jax
import jax.numpy as jnp
from jax.experimental import pallas as pl

E_BLK = 6400


def _edge_mlp_body(ef_ref, w1_ref, b1_ref, w2_ref, b2_ref, out_ref):
    a = jnp.dot(ef_ref[...], w1_ref[...], preferred_element_type=jnp.float32)
    a = jnp.maximum(a + b1_ref[...], 0.0)
    m = jnp.dot(a, w2_ref[...], preferred_element_type=jnp.float32)
    out_ref[...] = m + b2_ref[...]


def _edge_mlp(ef, W1p, b1, W2, b2):
    e = ef.shape[0]
    grid = e // E_BLK
    return pl.pallas_call(
        _edge_mlp_body,
        grid=(grid,),
        in_specs=[
            pl.BlockSpec((E_BLK, 8), lambda i: (i, 0)),
            pl.BlockSpec((8, 256), lambda i: (0, 0)),
            pl.BlockSpec((1, 256), lambda i: (0, 0)),
            pl.BlockSpec((256, 256), lambda i: (0, 0)),
            pl.BlockSpec((1, 256), lambda i: (0, 0)),
        ],
        out_specs=pl.BlockSpec((E_BLK, 256), lambda i: (i, 0)),
        out_shape=jax.ShapeDtypeStruct((e, 256), jnp.float32),
    )(ef, W1p, b1, W2, b2)


def _segment_max0(data, seg, num):
    out = jax.ops.segment_max(data, seg, num_segments=num)
    return jnp.where(jnp.isfinite(out), out, 0.0)


def _pointnet_layer(pos, src, dst, W1, b1, W2, b2, num_nodes):
    ef = jnp.concatenate(
        [pos[src], pos[src] - pos[dst], jnp.zeros((src.shape[0], 2), jnp.float32)],
        axis=-1)
    W1p = jnp.concatenate([W1, jnp.zeros((2, W1.shape[1]), jnp.float32)], axis=0)
    m = _edge_mlp(ef, W1p, b1[None, :], W2, b2[None, :])
    return _segment_max0(m, dst, num_nodes)


def _gcn(x, src, dst, W, b, num_nodes):
    loop = jnp.arange(num_nodes, dtype=src.dtype)
    s = jnp.concatenate([src, loop])
    d = jnp.concatenate([dst, loop])
    deg = jax.ops.segment_sum(jnp.ones(s.shape[0], x.dtype), d, num_segments=num_nodes)
    dinv = jnp.where(deg > 0, 1.0 / jnp.sqrt(deg), 0.0)
    norm = dinv[s] * dinv[d]
    xw = x @ W
    out = jax.ops.segment_sum(norm[:, None] * xw[s], d, num_segments=num_nodes)
    return out + b


def kernel(pos_0, edge_index_0, batch_0, pos_1, edge_index_1, batch_1,
           W1_0, b1_0, W2_0, b2_0, W1_1, b1_1, W2_1, b2_1,
           Wg0, bg0, Wg1, bg1, W3, b3, Wc, bc):
    n = pos_0.shape[0]
    s0, d0 = edge_index_0[0], edge_index_0[1]
    s1, d1 = edge_index_1[0], edge_index_1[1]
    h0 = jax.nn.relu(_pointnet_layer(pos_0, s0, d0, W1_0, b1_0, W2_0, b2_0, n))
    h1 = jax.nn.relu(_pointnet_layer(pos_1, s1, d1, W1_1, b1_1, W2_1, b2_1, n))
    h = jnp.concatenate([h0, h1], axis=1)
    g0 = jax.nn.relu(_gcn(h, s0, d0, Wg0, bg0, n))
    g1 = jax.nn.relu(_gcn(h, s1, d1, Wg1, bg1, n))
    h = jnp.concatenate([g0, g1], axis=1)
    h = jax.nn.relu(h @ W3 + b3)
    h = jax.nn.relu(h.reshape(-1))
    return h @ Wc + bc


# speedup vs baseline: 1.7716x; 1.7716x over previous
import functools

import jax
import jax.numpy as jnp
from jax import lax
from jax.experimental import pallas as pl
from jax.experimental.pallas import tpu as pltpu
from jax.experimental.pallas import tpu_sc as plsc

E_BLK = 6400

NW = 32
NODE_BLK = 313
NPAD = NW * NODE_BLK
CPAD = 336
DW = 2000
DCH = DW // 16
GB = 64
F = 256


def _segmax_body(m_hbm, dst_hbm, out_hbm, cnt_hbm,
                 dwin, idxbuf, locbuf, rows, accf, cntloc, sem):
    wid = lax.axis_index("s") * 2 + lax.axis_index("c")
    lo = wid * NODE_BLK
    neg = jnp.float32(-3.402823e38)

    def fill_acc(i, _):
        accf[pl.ds(i * 16, 16)] = jnp.full((16,), neg, jnp.float32)
        return 0
    lax.fori_loop(0, NODE_BLK * F // 16, fill_acc, 0)

    def fill_cnt(i, _):
        cntloc[pl.ds(i * 16, 16)] = jnp.zeros((16,), jnp.int32)
        return 0
    lax.fori_loop(0, CPAD // 16, fill_cnt, 0)

    def fill_idx(i, _):
        idxbuf[pl.ds(i * 16, 16)] = jnp.zeros((16,), jnp.int32)
        locbuf[pl.ds(i * 16, 16)] = jnp.zeros((16,), jnp.int32)
        return 0
    lax.fori_loop(0, (DW + 32) // 16, fill_idx, 0)

    iota16 = lax.iota(jnp.int32, 16)
    lo_v = jnp.full((16,), lo, jnp.int32)
    hi_v = jnp.full((16,), lo + NODE_BLK, jnp.int32)

    def psum16(x):
        for s in (1, 2, 4, 8):
            idx = jnp.maximum(iota16 - s, 0)
            g = x.at[idx].get(mode="promise_in_bounds")
            x = x + jnp.where(iota16 >= s, g, 0).astype(jnp.int32)
        return x

    def window(w, _):
        pltpu.sync_copy(dst_hbm.at[pl.ds(w * DW, DW)], dwin)

        def chunk(k, cnt):
            d16 = dwin[pl.ds(k * 16, 16)]
            msk = (d16 >= lo_v) & (d16 < hi_v)
            mi = jnp.where(msk, 1, 0).astype(jnp.int32)
            cum = psum16(mi)
            pos = cum + (cnt - 1)
            eidx = jnp.full((16,), w * DW + k * 16, jnp.int32) + iota16
            plsc.store_scatter(idxbuf, [pos], eidx, mask=msk)
            plsc.store_scatter(locbuf, [pos], d16 - lo_v, mask=msk)
            return cnt + cum[15]
        cnt = lax.fori_loop(0, DCH, chunk, jnp.int32(0))

        def batch(b, _):
            base = b * GB
            pltpu.async_copy(m_hbm.at[idxbuf.at[pl.ds(base, GB)]],
                             rows, sem).wait()
            nrows = jnp.minimum(cnt - base, GB)

            def one(i, _):
                r = locbuf[pl.ds(base + i, 16)][0]
                cv = cntloc[pl.ds(r, 16)]
                cntloc[pl.ds(r, 16)] = cv + jnp.where(
                    iota16 == 0, 1, 0).astype(jnp.int32)
                off = r * F
                for j in range(F // 16):
                    a = accf[pl.ds(off + j * 16, 16)]
                    v = rows[i, pl.ds(j * 16, 16)]
                    accf[pl.ds(off + j * 16, 16)] = jnp.maximum(a, v)
                return 0
            lax.fori_loop(0, nrows, one, 0)
            return 0
        lax.fori_loop(0, lax.shift_right_logical(cnt + GB - 1, 6), batch, 0)
        return 0
    lax.fori_loop(0, dst_hbm.shape[0] // DW, window, 0)

    pltpu.sync_copy(accf, out_hbm.at[pl.ds(lo * F, NODE_BLK * F)])
    pltpu.sync_copy(cntloc, cnt_hbm.at[pl.ds(wid * CPAD, CPAD)])


@jax.jit
def _segmax_sc(m, dst):
    mesh = plsc.VectorSubcoreMesh(core_axis_name="c", subcore_axis_name="s")
    e = dst.shape[0]
    fn = functools.partial(
        pl.kernel,
        out_type=[
            jax.ShapeDtypeStruct((NPAD * F,), jnp.float32),
            jax.ShapeDtypeStruct((NW * CPAD,), jnp.int32),
        ],
        mesh=mesh,
        scratch_types=[
            pltpu.VMEM((DW,), jnp.int32),
            pltpu.VMEM((DW + 32,), jnp.int32),
            pltpu.VMEM((DW + 32,), jnp.int32),
            pltpu.VMEM((GB, F), jnp.float32),
            pltpu.VMEM((NODE_BLK * F,), jnp.float32),
            pltpu.VMEM((CPAD,), jnp.int32),
            pltpu.SemaphoreType.DMA,
        ],
        compiler_params=pltpu.CompilerParams(needs_layout_passes=False),
    )(_segmax_body)
    out_flat, cnt_flat = fn(m, dst)
    out = out_flat.reshape(NPAD, F)
    cnt = cnt_flat.reshape(NW, CPAD)[:, :NODE_BLK].reshape(NPAD)
    return out, cnt


G = 128
EPW = 5000
GB2 = 40
NB2 = EPW // GB2
NPAD2 = 10016
NPAD3 = 10112


def _gcn_body(xw_hbm, src_hbm, dst2_hbm, dflat_hbm, dinv_hbm, out_hbm,
              swin, dwin2, dflat, dinv_v, rows, normbuf, shared, sem):
    cid = lax.axis_index("c")
    sid = lax.axis_index("s")
    wid = sid * 2 + cid

    pltpu.sync_copy(dinv_hbm, dinv_v)
    pltpu.sync_copy(src_hbm.at[pl.ds(wid * EPW, EPW)], swin.at[pl.ds(0, EPW)])
    pltpu.sync_copy(dst2_hbm.at[wid], dwin2)
    pltpu.sync_copy(dflat_hbm.at[pl.ds(wid * EPW, EPW)], dflat.at[pl.ds(0, EPW)])
    for t in range(2):
        swin[pl.ds(EPW + t * 16, 16)] = jnp.zeros((16,), jnp.int32)
        dflat[pl.ds(EPW + t * 16, 16)] = jnp.zeros((16,), jnp.int32)

    rows2 = rows

    def zfill(i, _):
        for j in range(G // 16):
            rows2[i, pl.ds(j * 16, 16)] = jnp.zeros((16,), jnp.float32)
        return 0
    lax.fori_loop(0, GB2, zfill, 0)
    stripe = NPAD3 // 16
    rbase = sid * stripe
    for k in range(stripe // GB2):
        pltpu.sync_copy(rows2, shared.at[pl.ds(rbase + k * GB2, GB2), :])
    rem = stripe - (stripe // GB2) * GB2
    pltpu.sync_copy(rows2.at[pl.ds(0, rem), :],
                    shared.at[pl.ds(rbase + stripe - rem, rem), :])
    plsc.subcore_barrier()

    def batch(b, _):
        base = b * GB2
        pltpu.async_copy(xw_hbm.at[swin.at[pl.ds(base, GB2)]], rows2,
                         sem).wait()
        for c in range((GB2 + 15) // 16):
            off = base + c * 16
            s16 = swin[pl.ds(off, 16)]
            d16 = dflat[pl.ds(off, 16)]
            sg = plsc.load_gather(dinv_v, [s16])
            dg = plsc.load_gather(dinv_v, [d16])
            normbuf[pl.ds(c * 16, 16)] = sg * dg

        def one(i, _):
            nv = normbuf[pl.ds(i, 16)][0]
            for j in range(G // 16):
                rows2[i, pl.ds(j * 16, 16)] = rows2[i, pl.ds(j * 16, 16)] * nv
            return 0
        lax.fori_loop(0, GB2, one, 0)
        pltpu.sync_copy(rows2, shared.at[dwin2.at[b]], add=True)
        return 0

    lax.fori_loop(0, NB2, batch, 0)
    plsc.subcore_barrier()
    pltpu.sync_copy(shared.at[pl.ds(rbase, stripe), :],
                    out_hbm.at[cid, pl.ds(rbase, stripe), :])


@jax.jit
def _gcn_aggr_sc(xw, src, dst, dinv):
    mesh = plsc.VectorSubcoreMesh(core_axis_name="c", subcore_axis_name="s")
    xwp = jnp.zeros((NPAD2, G), jnp.float32).at[:xw.shape[0]].set(xw)
    dinvp = jnp.zeros((NPAD2,), jnp.float32).at[:dinv.shape[0]].set(dinv)
    fn = functools.partial(
        pl.kernel,
        out_type=jax.ShapeDtypeStruct((2, NPAD3, G), jnp.float32),
        mesh=mesh,
        scratch_types=[
            pltpu.VMEM((EPW + 32,), jnp.int32),
            pltpu.VMEM((NB2, GB2), jnp.int32),
            pltpu.VMEM((EPW + 32,), jnp.int32),
            pltpu.VMEM((NPAD2,), jnp.float32),
            pltpu.VMEM((GB2, G), jnp.float32),
            pltpu.VMEM((64,), jnp.float32),
            pltpu.VMEM_SHARED((NPAD3, G), jnp.float32),
            pltpu.SemaphoreType.DMA,
        ],
        compiler_params=pltpu.CompilerParams(needs_layout_passes=False),
    )(_gcn_body)
    return fn(xwp, src, dst.reshape(NW, NB2, GB2), dst, dinvp)


EFW = 1250


def _ef_body(px_hbm, py_hbm, pz_hbm, src_hbm, dst_hbm, ef_hbm,
             px, py, pz, swin, dwin, efstage):
    cid = lax.axis_index("c")
    sid = lax.axis_index("s")
    wid = sid * 2 + cid
    iota16 = lax.iota(jnp.int32, 16)

    pltpu.sync_copy(px_hbm, px)
    pltpu.sync_copy(py_hbm, py)
    pltpu.sync_copy(pz_hbm, pz)
    pltpu.sync_copy(src_hbm.at[pl.ds(wid * EPW, EPW)], swin)
    pltpu.sync_copy(dst_hbm.at[pl.ds(wid * EPW, EPW)], dwin)

    def zfill(i, _):
        efstage[pl.ds(i * 16, 16)] = jnp.zeros((16,), jnp.float32)
        return 0
    lax.fori_loop(0, EFW * 8 // 16, zfill, 0)

    def window(w, _):
        def chunk(k, _):
            off = w * EFW + k * 16
            s16 = swin[pl.ds(off, 16)]
            d16 = dwin[pl.ds(off, 16)]
            sx = plsc.load_gather(px, [s16])
            sy = plsc.load_gather(py, [s16])
            sz = plsc.load_gather(pz, [s16])
            tx = plsc.load_gather(px, [d16])
            ty = plsc.load_gather(py, [d16])
            tz = plsc.load_gather(pz, [d16])
            pos0 = (jnp.full((16,), k * 16, jnp.int32) + iota16) * 8
            plsc.store_scatter(efstage, [pos0], sx)
            plsc.store_scatter(efstage, [pos0 + 1], sy)
            plsc.store_scatter(efstage, [pos0 + 2], sz)
            plsc.store_scatter(efstage, [pos0 + 3], sx - tx)
            plsc.store_scatter(efstage, [pos0 + 4], sy - ty)
            plsc.store_scatter(efstage, [pos0 + 5], sz - tz)
            return 0
        lax.fori_loop(0, EFW // 16, chunk, 0)
        pltpu.sync_copy(
            efstage, ef_hbm.at[pl.ds((wid * EPW + w * EFW) * 8, EFW * 8)])
        return 0
    lax.fori_loop(0, EPW // EFW, window, 0)


@jax.jit
def _ef_build_sc(pos, src, dst):
    mesh = plsc.VectorSubcoreMesh(core_axis_name="c", subcore_axis_name="s")
    e = src.shape[0]
    posT = jnp.zeros((3, NPAD2), jnp.float32).at[:, :pos.shape[0]].set(pos.T)
    pxa, pya, pza = posT[0], posT[1], posT[2]
    fn = functools.partial(
        pl.kernel,
        out_type=jax.ShapeDtypeStruct((e * 8,), jnp.float32),
        mesh=mesh,
        scratch_types=[
            pltpu.VMEM((NPAD2,), jnp.float32),
            pltpu.VMEM((NPAD2,), jnp.float32),
            pltpu.VMEM((NPAD2,), jnp.float32),
            pltpu.VMEM((EPW,), jnp.int32),
            pltpu.VMEM((EPW,), jnp.int32),
            pltpu.VMEM((EFW * 8,), jnp.float32),
        ],
        compiler_params=pltpu.CompilerParams(needs_layout_passes=False),
    )(_ef_body)
    ef = fn(pxa, pya, pza, src, dst)
    return ef.reshape(e, 8)


def _edge_mlp_body(ef_ref, w1_ref, b1_ref, w2_ref, b2_ref, out_ref):
    a = jnp.dot(ef_ref[...], w1_ref[...], preferred_element_type=jnp.float32)
    a = jnp.maximum(a + b1_ref[...], 0.0)
    m = jnp.dot(a, w2_ref[...], preferred_element_type=jnp.float32)
    out_ref[...] = m + b2_ref[...]


def _edge_mlp(ef, W1p, b1, W2, b2):
    e = ef.shape[0]
    grid = e // E_BLK
    return pl.pallas_call(
        _edge_mlp_body,
        grid=(grid,),
        in_specs=[
            pl.BlockSpec((E_BLK, 8), lambda i: (i, 0)),
            pl.BlockSpec((8, 256), lambda i: (0, 0)),
            pl.BlockSpec((1, 256), lambda i: (0, 0)),
            pl.BlockSpec((256, 256), lambda i: (0, 0)),
            pl.BlockSpec((1, 256), lambda i: (0, 0)),
        ],
        out_specs=pl.BlockSpec((E_BLK, 256), lambda i: (i, 0)),
        out_shape=jax.ShapeDtypeStruct((e, 256), jnp.float32),
    )(ef, W1p, b1, W2, b2)


def _pointnet_layer(pos, src, dst, W1, b1, W2, b2, num_nodes):
    ef = _ef_build_sc(pos, src, dst)
    W1p = jnp.concatenate([W1, jnp.zeros((2, W1.shape[1]), jnp.float32)], axis=0)
    m = _edge_mlp(ef, W1p, b1[None, :], W2, b2[None, :])
    mx, cnt = _segmax_sc(m, dst)
    return jax.nn.relu(mx[:num_nodes]), cnt[:num_nodes]


def _gcn(x, src, dst, W, b, num_nodes, deg):
    dinv = 1.0 / jnp.sqrt(deg.astype(jnp.float32))
    xw = x @ W
    parts = _gcn_aggr_sc(xw, src, dst, dinv)
    out = (parts[0] + parts[1])[:num_nodes]
    out = out + dinv[:, None] * dinv[:, None] * xw
    return out + b


def kernel(pos_0, edge_index_0, batch_0, pos_1, edge_index_1, batch_1,
           W1_0, b1_0, W2_0, b2_0, W1_1, b1_1, W2_1, b2_1,
           Wg0, bg0, Wg1, bg1, W3, b3, Wc, bc):
    n = pos_0.shape[0]
    s0, d0 = edge_index_0[0], edge_index_0[1]
    s1, d1 = edge_index_1[0], edge_index_1[1]
    h0, cnt0 = _pointnet_layer(pos_0, s0, d0, W1_0, b1_0, W2_0, b2_0, n)
    h1, cnt1 = _pointnet_layer(pos_1, s1, d1, W1_1, b1_1, W2_1, b2_1, n)
    h = jnp.concatenate([h0, h1], axis=1)
    deg0 = cnt0 + 1
    deg1 = cnt1 + 1
    g0 = jax.nn.relu(_gcn(h, s0, d0, Wg0, bg0, n, deg0))
    g1 = jax.nn.relu(_gcn(h, s1, d1, Wg1, bg1, n, deg1))
    h = jnp.concatenate([g0, g1], axis=1)
    h = jax.nn.relu(h @ W3 + b3)
    h = jax.nn.relu(h.reshape(-1))
    return h @ Wc + bc


# speedup vs baseline: 1.8304x; 1.0332x over previous
import functools

import jax
import jax.numpy as jnp
from jax import lax
from jax.experimental import pallas as pl
from jax.experimental.pallas import tpu as pltpu
from jax.experimental.pallas import tpu_sc as plsc

E_BLK = 6400

NW = 32
NODE_BLK = 313
NPAD = NW * NODE_BLK
CPAD = 336
DW = 2000
DCH = DW // 16
GB = 64
F = 256


def _segmax_body(m_hbm, dst_hbm, out_hbm, cnt_hbm,
                 dwin, idxbuf, locbuf, rows, rowsb, accf, cntloc, sem, semb):
    wid = lax.axis_index("s") * 2 + lax.axis_index("c")
    lo = wid * NODE_BLK
    neg = jnp.float32(-3.402823e38)

    def fill_acc(i, _):
        accf[pl.ds(i * 16, 16)] = jnp.full((16,), neg, jnp.float32)
        return 0
    lax.fori_loop(0, NODE_BLK * F // 16, fill_acc, 0)

    def fill_cnt(i, _):
        cntloc[pl.ds(i * 16, 16)] = jnp.zeros((16,), jnp.int32)
        return 0
    lax.fori_loop(0, CPAD // 16, fill_cnt, 0)

    def fill_idx(i, _):
        idxbuf[pl.ds(i * 16, 16)] = jnp.zeros((16,), jnp.int32)
        locbuf[pl.ds(i * 16, 16)] = jnp.zeros((16,), jnp.int32)
        return 0
    lax.fori_loop(0, (DW + 32) // 16, fill_idx, 0)

    iota16 = lax.iota(jnp.int32, 16)
    lo_v = jnp.full((16,), lo, jnp.int32)
    hi_v = jnp.full((16,), lo + NODE_BLK, jnp.int32)

    def psum16(x):
        for s in (1, 2, 4, 8):
            idx = jnp.maximum(iota16 - s, 0)
            g = x.at[idx].get(mode="promise_in_bounds")
            x = x + jnp.where(iota16 >= s, g, 0).astype(jnp.int32)
        return x

    def window(w, _):
        pltpu.sync_copy(dst_hbm.at[pl.ds(w * DW, DW)], dwin)

        def chunk(k, cnt):
            d16 = dwin[pl.ds(k * 16, 16)]
            msk = (d16 >= lo_v) & (d16 < hi_v)
            eidx = jnp.full((16,), w * DW + k * 16, jnp.int32) + iota16
            plsc.store_compressed(idxbuf.at[pl.ds(cnt, 16)], eidx, mask=msk)
            plsc.store_compressed(locbuf.at[pl.ds(cnt, 16)], d16 - lo_v,
                                  mask=msk)
            return cnt + plsc.all_reduce_population_count(msk)[0]
        cnt = lax.fori_loop(0, DCH, chunk, jnp.int32(0))

        nb = lax.shift_right_logical(cnt + GB - 1, 6)

        def start(b, buf, sm):
            pltpu.async_copy(m_hbm.at[idxbuf.at[pl.ds(b * GB, GB)]],
                             buf, sm)

        def wait(buf, sm):
            pltpu.make_async_copy(m_hbm.at[idxbuf.at[pl.ds(0, GB)]],
                                  buf, sm).wait()

        def drain(base, buf):
            nrows = jnp.minimum(cnt - base, GB)
            one1 = jnp.where(iota16 == 0, 1, 0).astype(jnp.int32)
            r0 = locbuf[pl.ds(base, 16)][0]

            def one(i, rcur):
                rnxt = locbuf[pl.ds(base + i + 1, 16)][0]
                cv = cntloc[pl.ds(rcur, 16)]
                cntloc[pl.ds(rcur, 16)] = cv + one1
                off = rcur * F
                loads = []
                for j in range(F // 16):
                    loads.append((accf[pl.ds(off + j * 16, 16)],
                                  buf[i, pl.ds(j * 16, 16)]))
                for j in range(F // 16):
                    a, v = loads[j]
                    accf[pl.ds(off + j * 16, 16)] = jnp.maximum(a, v)
                return rnxt
            lax.fori_loop(0, nrows, one, r0)

        @pl.when(nb > 0)
        def _():
            start(0, rows, sem)

        def pair(t, _):
            b0 = t * 2

            @pl.when(b0 + 1 < nb)
            def _():
                start(b0 + 1, rowsb, semb)
            wait(rows, sem)
            drain(b0 * GB, rows)

            @pl.when(b0 + 1 < nb)
            def _():
                @pl.when(b0 + 2 < nb)
                def _():
                    start(b0 + 2, rows, sem)
                wait(rowsb, semb)
                drain((b0 + 1) * GB, rowsb)
            return 0
        lax.fori_loop(0, lax.shift_right_logical(nb + 1, 1), pair, 0)
        return 0
    lax.fori_loop(0, dst_hbm.shape[0] // DW, window, 0)

    pltpu.sync_copy(accf, out_hbm.at[pl.ds(lo * F, NODE_BLK * F)])
    pltpu.sync_copy(cntloc, cnt_hbm.at[pl.ds(wid * CPAD, CPAD)])


@jax.jit
def _segmax_sc(m, dst):
    mesh = plsc.VectorSubcoreMesh(core_axis_name="c", subcore_axis_name="s")
    e = dst.shape[0]
    fn = functools.partial(
        pl.kernel,
        out_type=[
            jax.ShapeDtypeStruct((NPAD * F,), jnp.float32),
            jax.ShapeDtypeStruct((NW * CPAD,), jnp.int32),
        ],
        mesh=mesh,
        scratch_types=[
            pltpu.VMEM((DW,), jnp.int32),
            pltpu.VMEM((DW + 32,), jnp.int32),
            pltpu.VMEM((DW + 32,), jnp.int32),
            pltpu.VMEM((GB, F), jnp.float32),
            pltpu.VMEM((GB, F), jnp.float32),
            pltpu.VMEM((NODE_BLK * F,), jnp.float32),
            pltpu.VMEM((CPAD,), jnp.int32),
            pltpu.SemaphoreType.DMA,
            pltpu.SemaphoreType.DMA,
        ],
        compiler_params=pltpu.CompilerParams(needs_layout_passes=False),
    )(_segmax_body)
    out_flat, cnt_flat = fn(m, dst)
    out = out_flat.reshape(NPAD, F)
    cnt = cnt_flat.reshape(NW, CPAD)[:, :NODE_BLK].reshape(NPAD)
    return out, cnt


G = 128
EPW = 5000
GB2 = 40
NB2 = EPW // GB2
NPAD2 = 10016
NPAD3 = 10112


def _gcn_body(xw_hbm, src_hbm, dst2_hbm, out_hbm,
              swin, dwin2, rows, shared, sem):
    cid = lax.axis_index("c")
    sid = lax.axis_index("s")
    wid = sid * 2 + cid

    pltpu.sync_copy(src_hbm.at[pl.ds(wid * EPW, EPW)], swin)
    pltpu.sync_copy(dst2_hbm.at[wid], dwin2)

    rows2 = rows

    def zfill(i, _):
        for j in range(G // 16):
            rows2[i, pl.ds(j * 16, 16)] = jnp.zeros((16,), jnp.float32)
        return 0
    lax.fori_loop(0, GB2, zfill, 0)
    stripe = NPAD3 // 16
    rbase = sid * stripe
    for k in range(stripe // GB2):
        pltpu.sync_copy(rows2, shared.at[pl.ds(rbase + k * GB2, GB2), :])
    rem = stripe - (stripe // GB2) * GB2
    pltpu.sync_copy(rows2.at[pl.ds(0, rem), :],
                    shared.at[pl.ds(rbase + stripe - rem, rem), :])
    plsc.subcore_barrier()

    def batch(b, _):
        base = b * GB2
        pltpu.async_copy(xw_hbm.at[swin.at[pl.ds(base, GB2)]], rows2,
                         sem).wait()
        pltpu.sync_copy(rows2, shared.at[dwin2.at[b]], add=True)
        return 0

    lax.fori_loop(0, NB2, batch, 0)
    plsc.subcore_barrier()
    pltpu.sync_copy(shared.at[pl.ds(rbase, stripe), :],
                    out_hbm.at[cid, pl.ds(rbase, stripe), :])


@jax.jit
def _gcn_aggr_sc(xw2, src, dst):
    mesh = plsc.VectorSubcoreMesh(core_axis_name="c", subcore_axis_name="s")
    xwp = jnp.zeros((NPAD2, G), jnp.float32).at[:xw2.shape[0]].set(xw2)
    fn = functools.partial(
        pl.kernel,
        out_type=jax.ShapeDtypeStruct((2, NPAD3, G), jnp.float32),
        mesh=mesh,
        scratch_types=[
            pltpu.VMEM((EPW,), jnp.int32),
            pltpu.VMEM((NB2, GB2), jnp.int32),
            pltpu.VMEM((GB2, G), jnp.float32),
            pltpu.VMEM_SHARED((NPAD3, G), jnp.float32),
            pltpu.SemaphoreType.DMA,
        ],
        compiler_params=pltpu.CompilerParams(needs_layout_passes=False),
    )(_gcn_body)
    return fn(xwp, src, dst.reshape(NW, NB2, GB2))


EFW = 1250


def _ef_body(px_hbm, py_hbm, pz_hbm, src_hbm, dst_hbm, ef_hbm,
             px, py, pz, swin, dwin, efstage):
    cid = lax.axis_index("c")
    sid = lax.axis_index("s")
    wid = sid * 2 + cid
    iota16 = lax.iota(jnp.int32, 16)

    pltpu.sync_copy(px_hbm, px)
    pltpu.sync_copy(py_hbm, py)
    pltpu.sync_copy(pz_hbm, pz)
    pltpu.sync_copy(src_hbm.at[pl.ds(wid * EPW, EPW)], swin.at[pl.ds(0, EPW)])
    pltpu.sync_copy(dst_hbm.at[pl.ds(wid * EPW, EPW)], dwin.at[pl.ds(0, EPW)])
    for t in range(2):
        swin[pl.ds(EPW + t * 16, 16)] = jnp.zeros((16,), jnp.int32)
        dwin[pl.ds(EPW + t * 16, 16)] = jnp.zeros((16,), jnp.int32)

    def zfill(i, _):
        efstage[pl.ds(i * 16, 16)] = jnp.zeros((16,), jnp.float32)
        return 0
    lax.fori_loop(0, (EFW * 8 + 128) // 16, zfill, 0)

    def window(w, _):
        def chunk(k, _):
            off = w * EFW + k * 16
            s16 = swin[pl.ds(off, 16)]
            d16 = dwin[pl.ds(off, 16)]
            sx = plsc.load_gather(px, [s16])
            sy = plsc.load_gather(py, [s16])
            sz = plsc.load_gather(pz, [s16])
            tx = plsc.load_gather(px, [d16])
            ty = plsc.load_gather(py, [d16])
            tz = plsc.load_gather(pz, [d16])
            pos0 = (jnp.full((16,), k * 16, jnp.int32) + iota16) * 8
            plsc.store_scatter(efstage, [pos0], sx)
            plsc.store_scatter(efstage, [pos0 + 1], sy)
            plsc.store_scatter(efstage, [pos0 + 2], sz)
            plsc.store_scatter(efstage, [pos0 + 3], sx - tx)
            plsc.store_scatter(efstage, [pos0 + 4], sy - ty)
            plsc.store_scatter(efstage, [pos0 + 5], sz - tz)
            return 0
        lax.fori_loop(0, (EFW + 15) // 16, chunk, 0)
        pltpu.sync_copy(
            efstage.at[pl.ds(0, EFW * 8)],
            ef_hbm.at[pl.ds((wid * EPW + w * EFW) * 8, EFW * 8)])
        return 0
    lax.fori_loop(0, EPW // EFW, window, 0)


@jax.jit
def _ef_build_sc(pos, src, dst):
    mesh = plsc.VectorSubcoreMesh(core_axis_name="c", subcore_axis_name="s")
    e = src.shape[0]
    posT = jnp.zeros((3, NPAD2), jnp.float32).at[:, :pos.shape[0]].set(pos.T)
    pxa, pya, pza = posT[0], posT[1], posT[2]
    fn = functools.partial(
        pl.kernel,
        out_type=jax.ShapeDtypeStruct((e * 8,), jnp.float32),
        mesh=mesh,
        scratch_types=[
            pltpu.VMEM((NPAD2,), jnp.float32),
            pltpu.VMEM((NPAD2,), jnp.float32),
            pltpu.VMEM((NPAD2,), jnp.float32),
            pltpu.VMEM((EPW + 32,), jnp.int32),
            pltpu.VMEM((EPW + 32,), jnp.int32),
            pltpu.VMEM((EFW * 8 + 128,), jnp.float32),
        ],
        compiler_params=pltpu.CompilerParams(needs_layout_passes=False),
    )(_ef_body)
    ef = fn(pxa, pya, pza, src, dst)
    return ef.reshape(e, 8)


def _edge_mlp_body(ef_ref, w1_ref, b1_ref, w2_ref, b2_ref, out_ref):
    a = jnp.dot(ef_ref[...], w1_ref[...], preferred_element_type=jnp.float32)
    a = jnp.maximum(a + b1_ref[...], 0.0)
    m = jnp.dot(a, w2_ref[...], preferred_element_type=jnp.float32)
    out_ref[...] = m + b2_ref[...]


def _edge_mlp(ef, W1p, b1, W2, b2):
    e = ef.shape[0]
    grid = e // E_BLK
    return pl.pallas_call(
        _edge_mlp_body,
        grid=(grid,),
        in_specs=[
            pl.BlockSpec((E_BLK, 8), lambda i: (i, 0)),
            pl.BlockSpec((8, 256), lambda i: (0, 0)),
            pl.BlockSpec((1, 256), lambda i: (0, 0)),
            pl.BlockSpec((256, 256), lambda i: (0, 0)),
            pl.BlockSpec((1, 256), lambda i: (0, 0)),
        ],
        out_specs=pl.BlockSpec((E_BLK, 256), lambda i: (i, 0)),
        out_shape=jax.ShapeDtypeStruct((e, 256), jnp.float32),
    )(ef, W1p, b1, W2, b2)


def _pointnet_layer(pos, src, dst, W1, b1, W2, b2, num_nodes):
    ef = _ef_build_sc(pos, src, dst)
    W1p = jnp.concatenate([W1, jnp.zeros((2, W1.shape[1]), jnp.float32)], axis=0)
    m = _edge_mlp(ef, W1p, b1[None, :], W2, b2[None, :])
    mx, cnt = _segmax_sc(m, dst)
    return jax.nn.relu(mx[:num_nodes]), cnt[:num_nodes]


def _gcn(x, src, dst, W, b, num_nodes, deg):
    dinv = 1.0 / jnp.sqrt(deg.astype(jnp.float32))
    xw = x @ W
    parts = _gcn_aggr_sc(dinv[:, None] * xw, src, dst)
    out = dinv[:, None] * (parts[0] + parts[1])[:num_nodes]
    out = out + dinv[:, None] * dinv[:, None] * xw
    return out + b


def kernel(pos_0, edge_index_0, batch_0, pos_1, edge_index_1, batch_1,
           W1_0, b1_0, W2_0, b2_0, W1_1, b1_1, W2_1, b2_1,
           Wg0, bg0, Wg1, bg1, W3, b3, Wc, bc):
    n = pos_0.shape[0]
    s0, d0 = edge_index_0[0], edge_index_0[1]
    s1, d1 = edge_index_1[0], edge_index_1[1]
    h0, cnt0 = _pointnet_layer(pos_0, s0, d0, W1_0, b1_0, W2_0, b2_0, n)
    h1, cnt1 = _pointnet_layer(pos_1, s1, d1, W1_1, b1_1, W2_1, b2_1, n)
    h = jnp.concatenate([h0, h1], axis=1)
    deg0 = cnt0 + 1
    deg1 = cnt1 + 1
    g0 = jax.nn.relu(_gcn(h, s0, d0, Wg0, bg0, n, deg0))
    g1 = jax.nn.relu(_gcn(h, s1, d1, Wg1, bg1, n, deg1))
    h = jnp.concatenate([g0, g1], axis=1)
    h = jax.nn.relu(h @ W3 + b3)
    h = jax.nn.relu(h.reshape(-1))
    return h @ Wc + bc
